# baseline (speedup 1.0000x reference)
import jax
import jax.numpy as jnp
from jax.experimental import pallas as pl
from jax.experimental.pallas import tpu as pltpu

_P = 4194304
_CHUNK = 1 << 20
_NCHUNK = _P // _CHUNK
_TOTAL = 2 * _NCHUNK
_NBUF = 5
_LOOK = _NBUF - 1


def _body(a_smem, b_any, out_vmem, inb, sems):
    bi = pl.program_id(0)
    j = pl.program_id(1)
    i = bi * _NCHUNK + j

    def in_copy(b_idx, j_idx, slot):
        return pltpu.make_async_copy(
            b_any.at[b_idx, pl.ds(0, 1), pl.ds(j_idx * _CHUNK, _CHUNK)],
            inb.at[slot],
            sems.at[slot],
        )

    @pl.when(i == 0)
    def _():
        for c in range(min(_LOOK, _TOTAL)):
            cb, cj = divmod(c, _NCHUNK)

            @pl.when(a_smem[cb] != 0.0)
            def _():
                in_copy(cb, cj, c % _NBUF).start()

    i2 = i + _LOOK
    b2 = jnp.minimum(i2 // _NCHUNK, 1)
    j2 = i2 % _NCHUNK

    @pl.when(jnp.logical_and(i2 < _TOTAL, a_smem[b2] != 0.0))
    def _():
        in_copy(b2, j2, i2 % _NBUF).start()

    a = a_smem[bi]

    @pl.when(a != 0.0)
    def _():
        in_copy(bi, j, i % _NBUF).wait()
        out_vmem[0] = a * inb[i % _NBUF]

    @pl.when(a == 0.0)
    def _():
        out_vmem[0] = jnp.zeros((1, _CHUNK), jnp.float32)


def kernel(B, A):
    a2 = A.reshape(2)
    out = pl.pallas_call(
        _body,
        grid=(2, _NCHUNK),
        in_specs=[
            pl.BlockSpec(memory_space=pltpu.SMEM),
            pl.BlockSpec(memory_space=pl.ANY),
        ],
        out_specs=pl.BlockSpec((1, 1, _CHUNK), lambda b, j: (b, 0, j)),
        out_shape=jax.ShapeDtypeStruct((2, 1, _P), jnp.float32),
        scratch_shapes=[
            pltpu.VMEM((_NBUF, 1, _CHUNK), jnp.float32),
            pltpu.SemaphoreType.DMA((_NBUF,)),
        ],
    )(a2, B)
    return out
